# baseline (speedup 1.0000x reference)
import functools

import jax
import jax.numpy as jnp
from jax import lax
from jax.experimental import pallas as pl
from jax.experimental.pallas import tpu as pltpu
from jax.experimental.pallas import tpu_sc as plsc

N = 10000
E = 320000
D = 128

CHUNK = 128
NC, NS = 2, 16
NW = NC * NS
CPT = 80
HCPT = CPT // 2
EP = NW * CPT * CHUNK
ZCHUNK = 80
NZ = N // ZCHUNK
ZITERS = (NZ + NS - 1) // NS


def _mm_kernel(x_ref, w_ref, b_ref, o_ref):
    o_ref[...] = (
        jnp.dot(x_ref[...], w_ref[...], preferred_element_type=jnp.float32)
        + b_ref[...]
    )


def _matmul(x, W, b):
    BN = 2000
    return pl.pallas_call(
        _mm_kernel,
        grid=(N // BN,),
        in_specs=[
            pl.BlockSpec((BN, D), lambda i: (i, 0)),
            pl.BlockSpec((D, D), lambda i: (0, 0)),
            pl.BlockSpec((1, D), lambda i: (0, 0)),
        ],
        out_specs=pl.BlockSpec((BN, D), lambda i: (i, 0)),
        out_shape=jax.ShapeDtypeStruct((N, D), jnp.float32),
    )(x, W, b.reshape(1, D))


def _ep_kernel(p_ref, x_ref, o_ref):
    o_ref[...] = jnp.maximum(p_ref[0] + p_ref[1], 0.0) + x_ref[...]


def _epilogue(partial, x):
    BN = 2000
    return pl.pallas_call(
        _ep_kernel,
        grid=(N // BN,),
        in_specs=[
            pl.BlockSpec((2, BN, D), lambda i: (0, i, 0)),
            pl.BlockSpec((BN, D), lambda i: (i, 0)),
        ],
        out_specs=pl.BlockSpec((BN, D), lambda i: (i, 0)),
        out_shape=jax.ShapeDtypeStruct((N, D), jnp.float32),
    )(partial, x)


def _scale_rows(rows, wbuf):
    def scale(g, c2):
        w16 = wbuf[pl.ds(g * 16, 16)]
        for k in range(16):
            s = w16[k]
            r = g * 16 + k
            for j in range(D // 16):
                sl = pl.ds(16 * j, 16)
                rows[r, sl] = rows[r, sl] * s
        return c2

    lax.fori_loop(0, CHUNK // 16, scale, 0)


def _edge_body(hidden_hbm, src_hbm, dst_hbm, w_hbm, partial_hbm,
               srcA, srcB, dstA, dstB, wA, wB, rowsA, rowsB, acc,
               semA, semB):
    cid = lax.axis_index("c")
    sid = lax.axis_index("s")
    wid = sid * NC + cid

    zero = jnp.zeros((16,), jnp.float32)

    def zrow(r, carry):
        for j in range(D // 16):
            rowsA[r, pl.ds(16 * j, 16)] = zero
        return carry

    lax.fori_loop(0, ZCHUNK, zrow, 0)
    for k in range(ZITERS):
        c = sid + NS * k

        @pl.when(c < NZ)
        def _():
            pltpu.sync_copy(
                rowsA.at[pl.ds(0, ZCHUNK)],
                acc.at[pl.ds(c * ZCHUNK, ZCHUNK)],
            )

    plsc.subcore_barrier()

    def load_idx(chunk, src_v, dst_v, w_v):
        base = chunk * CHUNK
        pltpu.sync_copy(src_hbm.at[pl.ds(base, CHUNK)], src_v)
        pltpu.sync_copy(dst_hbm.at[pl.ds(base, CHUNK)], dst_v.at[0])
        pltpu.sync_copy(w_hbm.at[pl.ds(base, CHUNK)], w_v)

    def body(g, carry):
        c0 = wid + NW * (2 * g)
        c1 = wid + NW * (2 * g + 1)
        load_idx(c0, srcA, dstA, wA)
        load_idx(c1, srcB, dstB, wB)
        cpA = pltpu.async_copy(hidden_hbm.at[srcA], rowsA, semA)
        cpB = pltpu.async_copy(hidden_hbm.at[srcB], rowsB, semB)
        cpA.wait()
        _scale_rows(rowsA, wA)
        pltpu.sync_copy(rowsA, acc.at[dstA.at[0]], add=True)
        cpB.wait()
        _scale_rows(rowsB, wB)
        pltpu.sync_copy(rowsB, acc.at[dstB.at[0]], add=True)
        return carry

    lax.fori_loop(0, CPT // 2, body, 0)
    plsc.subcore_barrier()

    for k in range(ZITERS):
        c = sid + NS * k

        @pl.when(c < NZ)
        def _():
            pltpu.sync_copy(
                acc.at[pl.ds(c * ZCHUNK, ZCHUNK)],
                partial_hbm.at[cid, pl.ds(c * ZCHUNK, ZCHUNK)],
            )


def _edge_pass(hidden, src, dst, w):
    mesh = plsc.VectorSubcoreMesh(core_axis_name="c", subcore_axis_name="s")
    f = functools.partial(
        pl.kernel,
        mesh=mesh,
        out_type=jax.ShapeDtypeStruct((NC, N, D), jnp.float32),
        scratch_types=[
            pltpu.VMEM((CHUNK,), jnp.int32),
            pltpu.VMEM((CHUNK,), jnp.int32),
            pltpu.VMEM((1, CHUNK), jnp.int32),
            pltpu.VMEM((1, CHUNK), jnp.int32),
            pltpu.VMEM((CHUNK,), jnp.float32),
            pltpu.VMEM((CHUNK,), jnp.float32),
            pltpu.VMEM((CHUNK, D), jnp.float32),
            pltpu.VMEM((CHUNK, D), jnp.float32),
            pltpu.VMEM_SHARED((N, D), jnp.float32),
            pltpu.SemaphoreType.DMA,
            pltpu.SemaphoreType.DMA,
        ],
    )(_edge_body)
    return f(hidden, src, dst, w)


def kernel(x, edge_index, edge_weight, W, b):
    hidden = _matmul(x, W, b)
    pad = EP - E
    src = jnp.concatenate([edge_index[0], jnp.zeros((pad,), jnp.int32)])
    dst = jnp.concatenate([edge_index[1], jnp.zeros((pad,), jnp.int32)])
    w = jnp.concatenate([edge_weight, jnp.zeros((pad,), jnp.float32)])
    partial = _edge_pass(hidden, src, dst, w)
    return _epilogue(partial, x)


# speedup vs baseline: 1.7617x; 1.7617x over previous
import functools

import jax
import jax.numpy as jnp
from jax import lax
from jax.experimental import pallas as pl
from jax.experimental.pallas import tpu as pltpu
from jax.experimental.pallas import tpu_sc as plsc

N = 10000
E = 320000
D = 128

CHUNK = 128
NC, NS = 2, 16
NW = NC * NS
CPT = 80
HCPT = CPT // 2
EP = NW * CPT * CHUNK
ZCHUNK = 80
NZ = N // ZCHUNK
ZITERS = (NZ + NS - 1) // NS


def _mm_kernel(x_ref, w_ref, b_ref, o_ref):
    o_ref[...] = (
        jnp.dot(x_ref[...], w_ref[...], preferred_element_type=jnp.float32)
        + b_ref[...]
    )


def _matmul(x, W, b):
    BN = 2000
    return pl.pallas_call(
        _mm_kernel,
        grid=(N // BN,),
        in_specs=[
            pl.BlockSpec((BN, D), lambda i: (i, 0)),
            pl.BlockSpec((D, D), lambda i: (0, 0)),
            pl.BlockSpec((1, D), lambda i: (0, 0)),
        ],
        out_specs=pl.BlockSpec((BN, D), lambda i: (i, 0)),
        out_shape=jax.ShapeDtypeStruct((N, D), jnp.float32),
    )(x, W, b.reshape(1, D))


def _ep_kernel(p_ref, x_ref, o_ref):
    o_ref[...] = jnp.maximum(p_ref[0] + p_ref[1], 0.0) + x_ref[...]


def _epilogue(partial, x):
    BN = 2000
    return pl.pallas_call(
        _ep_kernel,
        grid=(N // BN,),
        in_specs=[
            pl.BlockSpec((2, BN, D), lambda i: (0, i, 0)),
            pl.BlockSpec((BN, D), lambda i: (i, 0)),
        ],
        out_specs=pl.BlockSpec((BN, D), lambda i: (i, 0)),
        out_shape=jax.ShapeDtypeStruct((N, D), jnp.float32),
    )(partial, x)


def _scale_rows(rows, wbuf):
    def scale(g, c2):
        w16 = wbuf[pl.ds(g * 16, 16)]
        for k in range(16):
            s = w16[k]
            r = g * 16 + k
            for j in range(D // 16):
                sl = pl.ds(16 * j, 16)
                rows[r, sl] = rows[r, sl] * s
        return c2

    lax.fori_loop(0, CHUNK // 16, scale, 0)


def _edge_body(hidden_hbm, src_hbm, dst_hbm, w_hbm, partial_hbm,
               srcA, srcB, dstA, dstB, wA, wB, rowsA, rowsB, acc,
               semA, semB):
    cid = lax.axis_index("c")
    sid = lax.axis_index("s")
    wid = sid * NC + cid

    zero = jnp.zeros((16,), jnp.float32)

    def zrow(r, carry):
        for j in range(D // 16):
            rowsA[r, pl.ds(16 * j, 16)] = zero
        return carry

    lax.fori_loop(0, ZCHUNK, zrow, 0)
    for k in range(ZITERS):
        c = sid + NS * k

        @pl.when(c < NZ)
        def _():
            pltpu.sync_copy(
                rowsA.at[pl.ds(0, ZCHUNK)],
                acc.at[pl.ds(c * ZCHUNK, ZCHUNK)],
            )

    plsc.subcore_barrier()

    def load_idx(chunk, src_v, dst_v, w_v):
        base = chunk * CHUNK
        pltpu.sync_copy(src_hbm.at[pl.ds(base, CHUNK)], src_v)
        pltpu.sync_copy(dst_hbm.at[pl.ds(base, CHUNK)], dst_v.at[0])
        pltpu.sync_copy(w_hbm.at[pl.ds(base, CHUNK)], w_v)

    def body(g, carry):
        c0 = wid + NW * (2 * g)
        c1 = wid + NW * (2 * g + 1)
        load_idx(c0, srcA, dstA, wA)
        load_idx(c1, srcB, dstB, wB)
        cpA = pltpu.async_copy(hidden_hbm.at[srcA], rowsA, semA)
        cpB = pltpu.async_copy(hidden_hbm.at[srcB], rowsB, semB)
        cpA.wait()
        _scale_rows(rowsA, wA)
        pltpu.sync_copy(rowsA, acc.at[dstA.at[0]], add=True)
        cpB.wait()
        _scale_rows(rowsB, wB)
        pltpu.sync_copy(rowsB, acc.at[dstB.at[0]], add=True)
        return carry

    lax.fori_loop(0, CPT // 2, body, 0)
    plsc.subcore_barrier()

    for k in range(ZITERS):
        c = sid + NS * k

        @pl.when(c < NZ)
        def _():
            pltpu.sync_copy(
                acc.at[pl.ds(c * ZCHUNK, ZCHUNK)],
                partial_hbm.at[cid, pl.ds(c * ZCHUNK, ZCHUNK)],
            )


def _edge_pass(hidden, src, dst, w):
    mesh = plsc.VectorSubcoreMesh(core_axis_name="c", subcore_axis_name="s")
    f = functools.partial(
        pl.kernel,
        mesh=mesh,
        out_type=jax.ShapeDtypeStruct((NC, N, D), jnp.float32),
        scratch_types=[
            pltpu.VMEM((CHUNK,), jnp.int32),
            pltpu.VMEM((CHUNK,), jnp.int32),
            pltpu.VMEM((1, CHUNK), jnp.int32),
            pltpu.VMEM((1, CHUNK), jnp.int32),
            pltpu.VMEM((CHUNK,), jnp.float32),
            pltpu.VMEM((CHUNK,), jnp.float32),
            pltpu.VMEM((CHUNK, D), jnp.float32),
            pltpu.VMEM((CHUNK, D), jnp.float32),
            pltpu.VMEM_SHARED((N, D), jnp.float32),
            pltpu.SemaphoreType.DMA,
            pltpu.SemaphoreType.DMA,
        ],
    )(_edge_body)
    return f(hidden, src, dst, w)


def kernel(x, edge_index, edge_weight, W, b):
    hidden = _matmul(x, W, b)
    pad = EP - E
    spread = jnp.arange(pad, dtype=jnp.int32) % N
    src = jnp.concatenate([edge_index[0], spread])
    dst = jnp.concatenate([edge_index[1], spread])
    w = jnp.concatenate([edge_weight, jnp.zeros((pad,), jnp.float32)])
    partial = _edge_pass(hidden, src, dst, w)
    return _epilogue(partial, x)


# speedup vs baseline: 2.2643x; 1.2853x over previous
import functools

import jax
import jax.numpy as jnp
from jax import lax
from jax.experimental import pallas as pl
from jax.experimental.pallas import tpu as pltpu
from jax.experimental.pallas import tpu_sc as plsc

N = 10000
E = 320000
D = 128

CHUNK = 128
NC, NS = 2, 16
NW = NC * NS
CPT = 80
HCPT = CPT // 2
EP = NW * CPT * CHUNK
ZCHUNK = 80
NZ = N // ZCHUNK
ZITERS = (NZ + NS - 1) // NS


def _mm_kernel(x_ref, w_ref, b_ref, o_ref):
    o_ref[...] = (
        jnp.dot(x_ref[...], w_ref[...], preferred_element_type=jnp.float32)
        + b_ref[...]
    )


def _matmul(x, W, b):
    BN = 2000
    return pl.pallas_call(
        _mm_kernel,
        grid=(N // BN,),
        in_specs=[
            pl.BlockSpec((BN, D), lambda i: (i, 0)),
            pl.BlockSpec((D, D), lambda i: (0, 0)),
            pl.BlockSpec((1, D), lambda i: (0, 0)),
        ],
        out_specs=pl.BlockSpec((BN, D), lambda i: (i, 0)),
        out_shape=jax.ShapeDtypeStruct((N, D), jnp.float32),
    )(x, W, b.reshape(1, D))


def _ep_kernel(p_ref, x_ref, o_ref):
    o_ref[...] = jnp.maximum(p_ref[0] + p_ref[1], 0.0) + x_ref[...]


def _epilogue(partial, x):
    BN = 2000
    return pl.pallas_call(
        _ep_kernel,
        grid=(N // BN,),
        in_specs=[
            pl.BlockSpec((2, BN, D), lambda i: (0, i, 0)),
            pl.BlockSpec((BN, D), lambda i: (i, 0)),
        ],
        out_specs=pl.BlockSpec((BN, D), lambda i: (i, 0)),
        out_shape=jax.ShapeDtypeStruct((N, D), jnp.float32),
    )(partial, x)


def _scale_rows(rows, wbuf):
    def scale(g, c2):
        w16 = wbuf[pl.ds(g * 16, 16)]
        for k in range(16):
            s = w16[k]
            r = g * 16 + k
            for j in range(D // 16):
                sl = pl.ds(16 * j, 16)
                rows[r, sl] = rows[r, sl] * s
        return c2

    lax.fori_loop(0, CHUNK // 16, scale, 0)


def _edge_body(hidden_hbm, src_hbm, dst_hbm, w_hbm, partial_hbm,
               srcA, srcB, dstA, dstB, wA, wB, rowsA, rowsB, acc,
               semA, semB):
    cid = lax.axis_index("c")
    sid = lax.axis_index("s")
    wid = sid * NC + cid

    zero = jnp.zeros((16,), jnp.float32)

    def zrow(r, carry):
        for j in range(D // 16):
            rowsA[r, pl.ds(16 * j, 16)] = zero
        return carry

    lax.fori_loop(0, ZCHUNK, zrow, 0)
    for k in range(ZITERS):
        c = sid + NS * k

        @pl.when(c < NZ)
        def _():
            pltpu.sync_copy(
                rowsA.at[pl.ds(0, ZCHUNK)],
                acc.at[pl.ds(c * ZCHUNK, ZCHUNK)],
            )

    plsc.subcore_barrier()

    def load_idx(chunk, src_v, dst_v, w_v):
        base = chunk * CHUNK
        pltpu.sync_copy(src_hbm.at[pl.ds(base, CHUNK)], src_v)
        pltpu.sync_copy(dst_hbm.at[pl.ds(base, CHUNK)], dst_v.at[0])
        pltpu.sync_copy(w_hbm.at[pl.ds(base, CHUNK)], w_v)

    load_idx(wid, srcA, dstA, wA)
    load_idx(wid + NW, srcB, dstB, wB)
    pltpu.make_async_copy(hidden_hbm.at[srcA], rowsA, semA).start()

    def body(g, carry):
        c0 = wid + NW * (2 * g)
        c1 = wid + NW * (2 * g + 1)
        cpB = pltpu.async_copy(hidden_hbm.at[srcB], rowsB, semB)
        pltpu.make_async_copy(hidden_hbm.at[srcA], rowsA, semA).wait()
        _scale_rows(rowsA, wA)
        pltpu.sync_copy(rowsA, acc.at[dstA.at[0]], add=True)

        @pl.when(2 * g + 2 < CPT)
        def _():
            load_idx(c0 + 2 * NW, srcA, dstA, wA)
            pltpu.make_async_copy(hidden_hbm.at[srcA], rowsA, semA).start()

        cpB.wait()
        _scale_rows(rowsB, wB)
        pltpu.sync_copy(rowsB, acc.at[dstB.at[0]], add=True)

        @pl.when(2 * g + 3 < CPT)
        def _():
            load_idx(c1 + 2 * NW, srcB, dstB, wB)

        return carry

    lax.fori_loop(0, CPT // 2, body, 0)
    plsc.subcore_barrier()

    for k in range(ZITERS):
        c = sid + NS * k

        @pl.when(c < NZ)
        def _():
            pltpu.sync_copy(
                acc.at[pl.ds(c * ZCHUNK, ZCHUNK)],
                partial_hbm.at[cid, pl.ds(c * ZCHUNK, ZCHUNK)],
            )


def _edge_pass(hidden, src, dst, w):
    mesh = plsc.VectorSubcoreMesh(core_axis_name="c", subcore_axis_name="s")
    f = functools.partial(
        pl.kernel,
        mesh=mesh,
        out_type=jax.ShapeDtypeStruct((NC, N, D), jnp.float32),
        scratch_types=[
            pltpu.VMEM((CHUNK,), jnp.int32),
            pltpu.VMEM((CHUNK,), jnp.int32),
            pltpu.VMEM((1, CHUNK), jnp.int32),
            pltpu.VMEM((1, CHUNK), jnp.int32),
            pltpu.VMEM((CHUNK,), jnp.float32),
            pltpu.VMEM((CHUNK,), jnp.float32),
            pltpu.VMEM((CHUNK, D), jnp.float32),
            pltpu.VMEM((CHUNK, D), jnp.float32),
            pltpu.VMEM_SHARED((N, D), jnp.float32),
            pltpu.SemaphoreType.DMA,
            pltpu.SemaphoreType.DMA,
        ],
    )(_edge_body)
    return f(hidden, src, dst, w)


def kernel(x, edge_index, edge_weight, W, b):
    hidden = _matmul(x, W, b)
    pad = EP - E
    spread = jnp.arange(pad, dtype=jnp.int32) % N
    src = jnp.concatenate([edge_index[0], spread])
    dst = jnp.concatenate([edge_index[1], spread])
    w = jnp.concatenate([edge_weight, jnp.zeros((pad,), jnp.float32)])
    partial = _edge_pass(hidden, src, dst, w)
    return _epilogue(partial, x)


# speedup vs baseline: 2.4468x; 1.0806x over previous
import functools

import jax
import jax.numpy as jnp
from jax import lax
from jax.experimental import pallas as pl
from jax.experimental.pallas import tpu as pltpu
from jax.experimental.pallas import tpu_sc as plsc

N = 10000
E = 320000
D = 128

CHUNK = 128
NC, NS = 2, 16
NW = NC * NS
CPT = 80
HCPT = CPT // 2
EP = NW * CPT * CHUNK
ZCHUNK = 80
NZ = N // ZCHUNK
ZITERS = (NZ + NS - 1) // NS


def _mm_kernel(x_ref, w_ref, b_ref, o_ref):
    o_ref[...] = (
        jnp.dot(x_ref[...], w_ref[...], preferred_element_type=jnp.float32)
        + b_ref[...]
    )


def _matmul(x, W, b):
    BN = 2000
    return pl.pallas_call(
        _mm_kernel,
        grid=(N // BN,),
        in_specs=[
            pl.BlockSpec((BN, D), lambda i: (i, 0)),
            pl.BlockSpec((D, D), lambda i: (0, 0)),
            pl.BlockSpec((1, D), lambda i: (0, 0)),
        ],
        out_specs=pl.BlockSpec((BN, D), lambda i: (i, 0)),
        out_shape=jax.ShapeDtypeStruct((N, D), jnp.float32),
    )(x, W, b.reshape(1, D))


def _ep_kernel(p_ref, x_ref, o_ref):
    o_ref[...] = jnp.maximum(p_ref[0] + p_ref[1], 0.0) + x_ref[...]


def _epilogue(partial, x):
    BN = 2000
    return pl.pallas_call(
        _ep_kernel,
        grid=(N // BN,),
        in_specs=[
            pl.BlockSpec((2, BN, D), lambda i: (0, i, 0)),
            pl.BlockSpec((BN, D), lambda i: (i, 0)),
        ],
        out_specs=pl.BlockSpec((BN, D), lambda i: (i, 0)),
        out_shape=jax.ShapeDtypeStruct((N, D), jnp.float32),
    )(partial, x)


def _scale_rows(rows, wbuf):
    def scale(g, c2):
        w16 = wbuf[pl.ds(g * 16, 16)]
        for k in range(16):
            s = w16[k]
            r = g * 16 + k
            for j in range(D // 16):
                sl = pl.ds(16 * j, 16)
                rows[r, sl] = rows[r, sl] * s
        return c2

    lax.fori_loop(0, CHUNK // 16, scale, 0)


def _edge_body(hidden_hbm, src_hbm, dst_hbm, w_hbm, partial_hbm,
               srcA, srcB, dstA, dstB, wA, wB, rowsA, rowsB, acc,
               semA, semB, semSA, semSB):
    cid = lax.axis_index("c")
    sid = lax.axis_index("s")
    wid = sid * NC + cid

    zero = jnp.zeros((16,), jnp.float32)

    def zrow(r, carry):
        for j in range(D // 16):
            rowsA[r, pl.ds(16 * j, 16)] = zero
        return carry

    lax.fori_loop(0, ZCHUNK, zrow, 0)
    for k in range(ZITERS):
        c = sid + NS * k

        @pl.when(c < NZ)
        def _():
            pltpu.sync_copy(
                rowsA.at[pl.ds(0, ZCHUNK)],
                acc.at[pl.ds(c * ZCHUNK, ZCHUNK)],
            )

    plsc.subcore_barrier()

    def load_idx(chunk, src_v, dst_v, w_v):
        base = chunk * CHUNK
        pltpu.sync_copy(src_hbm.at[pl.ds(base, CHUNK)], src_v)
        pltpu.sync_copy(dst_hbm.at[pl.ds(base, CHUNK)], dst_v.at[0])
        pltpu.sync_copy(w_hbm.at[pl.ds(base, CHUNK)], w_v)

    load_idx(wid, srcA, dstA, wA)
    pltpu.make_async_copy(hidden_hbm.at[srcA], rowsA, semA).start()

    def body(g, carry):
        c0 = wid + NW * (2 * g)
        c1 = wid + NW * (2 * g + 1)

        @pl.when(g > 0)
        def _():
            pltpu.make_async_copy(rowsB, acc.at[dstB.at[0]], semSB).wait()

        load_idx(c1, srcB, dstB, wB)
        cpgB = pltpu.async_copy(hidden_hbm.at[srcB], rowsB, semB)
        pltpu.make_async_copy(hidden_hbm.at[srcA], rowsA, semA).wait()
        _scale_rows(rowsA, wA)
        cpsA = pltpu.async_copy(rowsA, acc.at[dstA.at[0]], semSA, add=True)
        cpgB.wait()
        _scale_rows(rowsB, wB)
        cpsA.wait()
        pltpu.async_copy(rowsB, acc.at[dstB.at[0]], semSB, add=True)

        @pl.when(2 * g + 2 < CPT)
        def _():
            load_idx(c0 + 2 * NW, srcA, dstA, wA)
            pltpu.make_async_copy(hidden_hbm.at[srcA], rowsA, semA).start()

        return carry

    lax.fori_loop(0, CPT // 2, body, 0)
    pltpu.make_async_copy(rowsB, acc.at[dstB.at[0]], semSB).wait()
    plsc.subcore_barrier()

    for k in range(ZITERS):
        c = sid + NS * k

        @pl.when(c < NZ)
        def _():
            pltpu.sync_copy(
                acc.at[pl.ds(c * ZCHUNK, ZCHUNK)],
                partial_hbm.at[cid, pl.ds(c * ZCHUNK, ZCHUNK)],
            )


def _edge_pass(hidden, src, dst, w):
    mesh = plsc.VectorSubcoreMesh(core_axis_name="c", subcore_axis_name="s")
    f = functools.partial(
        pl.kernel,
        mesh=mesh,
        out_type=jax.ShapeDtypeStruct((NC, N, D), jnp.float32),
        scratch_types=[
            pltpu.VMEM((CHUNK,), jnp.int32),
            pltpu.VMEM((CHUNK,), jnp.int32),
            pltpu.VMEM((1, CHUNK), jnp.int32),
            pltpu.VMEM((1, CHUNK), jnp.int32),
            pltpu.VMEM((CHUNK,), jnp.float32),
            pltpu.VMEM((CHUNK,), jnp.float32),
            pltpu.VMEM((CHUNK, D), jnp.float32),
            pltpu.VMEM((CHUNK, D), jnp.float32),
            pltpu.VMEM_SHARED((N, D), jnp.float32),
            pltpu.SemaphoreType.DMA,
            pltpu.SemaphoreType.DMA,
            pltpu.SemaphoreType.DMA,
            pltpu.SemaphoreType.DMA,
        ],
    )(_edge_body)
    return f(hidden, src, dst, w)


def kernel(x, edge_index, edge_weight, W, b):
    hidden = _matmul(x, W, b)
    pad = EP - E
    spread = jnp.arange(pad, dtype=jnp.int32) % N
    src = jnp.concatenate([edge_index[0], spread])
    dst = jnp.concatenate([edge_index[1], spread])
    w = jnp.concatenate([edge_weight, jnp.zeros((pad,), jnp.float32)])
    partial = _edge_pass(hidden, src, dst, w)
    return _epilogue(partial, x)


# speedup vs baseline: 3.0821x; 1.2596x over previous
import functools

import jax
import jax.numpy as jnp
from jax import lax
from jax.experimental import pallas as pl
from jax.experimental.pallas import tpu as pltpu
from jax.experimental.pallas import tpu_sc as plsc

N = 10000
E = 320000
D = 128

CHUNK = 128
NC, NS = 2, 16
NW = NC * NS
CPT = 80
HCPT = CPT // 2
EP = NW * CPT * CHUNK
ZCHUNK = 80
NZ = N // ZCHUNK
ZITERS = (NZ + NS - 1) // NS


def _mm_kernel(x_ref, w_ref, b_ref, o_ref):
    o_ref[...] = (
        jnp.dot(x_ref[...], w_ref[...], preferred_element_type=jnp.float32)
        + b_ref[...]
    )


def _matmul(x, W, b):
    BN = 2000
    return pl.pallas_call(
        _mm_kernel,
        grid=(N // BN,),
        in_specs=[
            pl.BlockSpec((BN, D), lambda i: (i, 0)),
            pl.BlockSpec((D, D), lambda i: (0, 0)),
            pl.BlockSpec((1, D), lambda i: (0, 0)),
        ],
        out_specs=pl.BlockSpec((BN, D), lambda i: (i, 0)),
        out_shape=jax.ShapeDtypeStruct((N, D), jnp.float32),
    )(x, W, b.reshape(1, D))


def _ep_kernel(p_ref, x_ref, o_ref):
    o_ref[...] = jnp.maximum(p_ref[0] + p_ref[1], 0.0) + x_ref[...]


def _epilogue(partial, x):
    BN = 2000
    return pl.pallas_call(
        _ep_kernel,
        grid=(N // BN,),
        in_specs=[
            pl.BlockSpec((2, BN, D), lambda i: (0, i, 0)),
            pl.BlockSpec((BN, D), lambda i: (i, 0)),
        ],
        out_specs=pl.BlockSpec((BN, D), lambda i: (i, 0)),
        out_shape=jax.ShapeDtypeStruct((N, D), jnp.float32),
    )(partial, x)


def _scale_rows(rows, wbuf):
    def scale(g, c2):
        w16 = wbuf[pl.ds(g * 16, 16)]
        for k in range(16):
            s = w16[k]
            r = g * 16 + k
            for j in range(D // 16):
                sl = pl.ds(16 * j, 16)
                rows[r, sl] = rows[r, sl] * s
        return c2

    lax.fori_loop(0, CHUNK // 16, scale, 0, unroll=2)


def _edge_body(hidden_hbm, src_hbm, dst_hbm, w_hbm, partial_hbm,
               srcA, srcB, dstA, dstB, wA, wB, rowsA, rowsB, acc,
               semA, semB, semSA, semSB):
    cid = lax.axis_index("c")
    sid = lax.axis_index("s")
    wid = sid * NC + cid

    zero = jnp.zeros((16,), jnp.float32)

    def zrow(r, carry):
        for j in range(D // 16):
            rowsA[r, pl.ds(16 * j, 16)] = zero
        return carry

    lax.fori_loop(0, ZCHUNK, zrow, 0)
    for k in range(ZITERS):
        c = sid + NS * k

        @pl.when(c < NZ)
        def _():
            pltpu.sync_copy(
                rowsA.at[pl.ds(0, ZCHUNK)],
                acc.at[pl.ds(c * ZCHUNK, ZCHUNK)],
            )

    plsc.subcore_barrier()

    def load_idx(chunk, src_v, dst_v, w_v, sem):
        base = chunk * CHUNK
        c1 = pltpu.async_copy(src_hbm.at[pl.ds(base, CHUNK)], src_v, sem)
        c2 = pltpu.async_copy(dst_hbm.at[pl.ds(base, CHUNK)], dst_v.at[0], sem)
        c3 = pltpu.async_copy(w_hbm.at[pl.ds(base, CHUNK)], w_v, sem)
        c1.wait()
        c2.wait()
        c3.wait()

    load_idx(wid, srcA, dstA, wA, semA)
    pltpu.make_async_copy(hidden_hbm.at[srcA], rowsA, semA).start()

    def body(g, carry):
        c0 = wid + NW * (2 * g)
        c1 = wid + NW * (2 * g + 1)

        @pl.when(g > 0)
        def _():
            pltpu.make_async_copy(rowsB, acc.at[dstB.at[0]], semSB).wait()

        load_idx(c1, srcB, dstB, wB, semB)
        cpgB = pltpu.async_copy(hidden_hbm.at[srcB], rowsB, semB)
        pltpu.make_async_copy(hidden_hbm.at[srcA], rowsA, semA).wait()
        _scale_rows(rowsA, wA)
        cpsA = pltpu.async_copy(rowsA, acc.at[dstA.at[0]], semSA, add=True)
        cpgB.wait()
        _scale_rows(rowsB, wB)
        cpsA.wait()
        pltpu.async_copy(rowsB, acc.at[dstB.at[0]], semSB, add=True)

        @pl.when(2 * g + 2 < CPT)
        def _():
            load_idx(c0 + 2 * NW, srcA, dstA, wA, semA)
            pltpu.make_async_copy(hidden_hbm.at[srcA], rowsA, semA).start()

        return carry

    lax.fori_loop(0, CPT // 2, body, 0)
    pltpu.make_async_copy(rowsB, acc.at[dstB.at[0]], semSB).wait()
    plsc.subcore_barrier()

    for k in range(ZITERS):
        c = sid + NS * k

        @pl.when(c < NZ)
        def _():
            pltpu.sync_copy(
                acc.at[pl.ds(c * ZCHUNK, ZCHUNK)],
                partial_hbm.at[cid, pl.ds(c * ZCHUNK, ZCHUNK)],
            )


def _edge_pass(hidden, src, dst, w):
    mesh = plsc.VectorSubcoreMesh(core_axis_name="c", subcore_axis_name="s")
    f = functools.partial(
        pl.kernel,
        mesh=mesh,
        out_type=jax.ShapeDtypeStruct((NC, N, D), jnp.float32),
        scratch_types=[
            pltpu.VMEM((CHUNK,), jnp.int32),
            pltpu.VMEM((CHUNK,), jnp.int32),
            pltpu.VMEM((1, CHUNK), jnp.int32),
            pltpu.VMEM((1, CHUNK), jnp.int32),
            pltpu.VMEM((CHUNK,), jnp.float32),
            pltpu.VMEM((CHUNK,), jnp.float32),
            pltpu.VMEM((CHUNK, D), jnp.float32),
            pltpu.VMEM((CHUNK, D), jnp.float32),
            pltpu.VMEM_SHARED((N, D), jnp.float32),
            pltpu.SemaphoreType.DMA,
            pltpu.SemaphoreType.DMA,
            pltpu.SemaphoreType.DMA,
            pltpu.SemaphoreType.DMA,
        ],
    )(_edge_body)
    return f(hidden, src, dst, w)


def kernel(x, edge_index, edge_weight, W, b):
    hidden = _matmul(x, W, b)
    pad = EP - E
    spread = jnp.arange(pad, dtype=jnp.int32) % N
    src = jnp.concatenate([edge_index[0], spread])
    dst = jnp.concatenate([edge_index[1], spread])
    w = jnp.concatenate([edge_weight, jnp.zeros((pad,), jnp.float32)])
    partial = _edge_pass(hidden, src, dst, w)
    return _epilogue(partial, x)


# speedup vs baseline: 3.5433x; 1.1496x over previous
import functools

import jax
import jax.numpy as jnp
from jax import lax
from jax.experimental import pallas as pl
from jax.experimental.pallas import tpu as pltpu
from jax.experimental.pallas import tpu_sc as plsc

N = 10000
E = 320000
D = 128

CHUNK = 128
NC, NS = 2, 16
NW = NC * NS
CPT = 81
EP = NW * CPT * CHUNK
ZCHUNK = 80
NZ = N // ZCHUNK
ZITERS = (NZ + NS - 1) // NS


def _mm_kernel(x_ref, w_ref, b_ref, o_ref):
    o_ref[...] = (
        jnp.dot(x_ref[...], w_ref[...], preferred_element_type=jnp.float32)
        + b_ref[...]
    )


def _matmul(x, W, b):
    BN = 2000
    return pl.pallas_call(
        _mm_kernel,
        grid=(N // BN,),
        in_specs=[
            pl.BlockSpec((BN, D), lambda i: (i, 0)),
            pl.BlockSpec((D, D), lambda i: (0, 0)),
            pl.BlockSpec((1, D), lambda i: (0, 0)),
        ],
        out_specs=pl.BlockSpec((BN, D), lambda i: (i, 0)),
        out_shape=jax.ShapeDtypeStruct((N, D), jnp.float32),
    )(x, W, b.reshape(1, D))


def _ep_kernel(p_ref, x_ref, o_ref):
    o_ref[...] = jnp.maximum(p_ref[0] + p_ref[1], 0.0) + x_ref[...]


def _epilogue(partial, x):
    BN = 2000
    return pl.pallas_call(
        _ep_kernel,
        grid=(N // BN,),
        in_specs=[
            pl.BlockSpec((2, BN, D), lambda i: (0, i, 0)),
            pl.BlockSpec((BN, D), lambda i: (i, 0)),
        ],
        out_specs=pl.BlockSpec((BN, D), lambda i: (i, 0)),
        out_shape=jax.ShapeDtypeStruct((N, D), jnp.float32),
    )(partial, x)


def _scale_rows(rows, wbuf):
    def scale(g, c2):
        w16 = wbuf[pl.ds(g * 16, 16)]
        for k in range(16):
            s = w16[k]
            r = g * 16 + k
            for j in range(D // 16):
                sl = pl.ds(16 * j, 16)
                rows[r, sl] = rows[r, sl] * s
        return c2

    lax.fori_loop(0, CHUNK // 16, scale, 0, unroll=2)


def _edge_body(hidden_hbm, src_hbm, dst_hbm, w_hbm, partial_hbm,
               srcA, srcB, srcC, dstA, dstB, dstC, wA, wB, wC,
               rowsA, rowsB, rowsC, acc,
               semA, semB, semC, semSA, semSB, semSC):
    cid = lax.axis_index("c")
    sid = lax.axis_index("s")
    wid = sid * NC + cid

    zero = jnp.zeros((16,), jnp.float32)

    def zrow(r, carry):
        for j in range(D // 16):
            rowsA[r, pl.ds(16 * j, 16)] = zero
        return carry

    lax.fori_loop(0, ZCHUNK, zrow, 0)
    for k in range(ZITERS):
        c = sid + NS * k

        @pl.when(c < NZ)
        def _():
            pltpu.sync_copy(
                rowsA.at[pl.ds(0, ZCHUNK)],
                acc.at[pl.ds(c * ZCHUNK, ZCHUNK)],
            )

    plsc.subcore_barrier()

    def load_idx(chunk, src_v, dst_v, w_v, sem):
        base = chunk * CHUNK
        c1 = pltpu.async_copy(src_hbm.at[pl.ds(base, CHUNK)], src_v, sem)
        c2 = pltpu.async_copy(dst_hbm.at[pl.ds(base, CHUNK)], dst_v.at[0], sem)
        c3 = pltpu.async_copy(w_hbm.at[pl.ds(base, CHUNK)], w_v, sem)
        c1.wait()
        c2.wait()
        c3.wait()

    bufs = ((srcA, dstA, wA, rowsA, semA, semSA),
            (srcB, dstB, wB, rowsB, semB, semSB),
            (srcC, dstC, wC, rowsC, semC, semSC))

    load_idx(wid, srcA, dstA, wA, semA)
    pltpu.make_async_copy(hidden_hbm.at[srcA], rowsA, semA).start()
    load_idx(wid + NW, srcB, dstB, wB, semB)
    pltpu.make_async_copy(hidden_hbm.at[srcB], rowsB, semB).start()

    def body(g, carry):
        for o in range(3):
            j = 3 * g + o
            src_v, dst_v, w_v, rows, sem, semS = bufs[o]
            psrc, pdst, pw, prows, psem, psemS = bufs[(o + 2) % 3]

            pltpu.make_async_copy(hidden_hbm.at[src_v], rows, sem).wait()
            _scale_rows(rows, w_v)
            pltpu.async_copy(rows, acc.at[dst_v.at[0]], semS, add=True)

            @pl.when(j + 2 < CPT)
            def _():
                def refill():
                    pltpu.make_async_copy(
                        prows, acc.at[pdst.at[0]], psemS).wait()
                    load_idx(wid + NW * (j + 2), psrc, pdst, pw, psem)
                    pltpu.make_async_copy(
                        hidden_hbm.at[psrc], prows, psem).start()

                if o == 0:
                    @pl.when(g > 0)
                    def _():
                        refill()

                    @pl.when(g == 0)
                    def _():
                        load_idx(wid + NW * (j + 2), psrc, pdst, pw, psem)
                        pltpu.make_async_copy(
                            hidden_hbm.at[psrc], prows, psem).start()
                else:
                    refill()

        return carry

    lax.fori_loop(0, CPT // 3, body, 0)
    for o in ((CPT - 3) % 3, (CPT - 2) % 3, (CPT - 1) % 3):
        _, dst_v, _, rows, _, semS = bufs[o]
        pltpu.make_async_copy(rows, acc.at[dst_v.at[0]], semS).wait()
    plsc.subcore_barrier()

    for k in range(ZITERS):
        c = sid + NS * k

        @pl.when(c < NZ)
        def _():
            pltpu.sync_copy(
                acc.at[pl.ds(c * ZCHUNK, ZCHUNK)],
                partial_hbm.at[cid, pl.ds(c * ZCHUNK, ZCHUNK)],
            )


def _edge_pass(hidden, src, dst, w):
    mesh = plsc.VectorSubcoreMesh(core_axis_name="c", subcore_axis_name="s")
    f = functools.partial(
        pl.kernel,
        mesh=mesh,
        out_type=jax.ShapeDtypeStruct((NC, N, D), jnp.float32),
        scratch_types=[
            pltpu.VMEM((CHUNK,), jnp.int32),
            pltpu.VMEM((CHUNK,), jnp.int32),
            pltpu.VMEM((CHUNK,), jnp.int32),
            pltpu.VMEM((1, CHUNK), jnp.int32),
            pltpu.VMEM((1, CHUNK), jnp.int32),
            pltpu.VMEM((1, CHUNK), jnp.int32),
            pltpu.VMEM((CHUNK,), jnp.float32),
            pltpu.VMEM((CHUNK,), jnp.float32),
            pltpu.VMEM((CHUNK,), jnp.float32),
            pltpu.VMEM((CHUNK, D), jnp.float32),
            pltpu.VMEM((CHUNK, D), jnp.float32),
            pltpu.VMEM((CHUNK, D), jnp.float32),
            pltpu.VMEM_SHARED((N, D), jnp.float32),
            pltpu.SemaphoreType.DMA,
            pltpu.SemaphoreType.DMA,
            pltpu.SemaphoreType.DMA,
            pltpu.SemaphoreType.DMA,
            pltpu.SemaphoreType.DMA,
            pltpu.SemaphoreType.DMA,
        ],
    )(_edge_body)
    return f(hidden, src, dst, w)


def kernel(x, edge_index, edge_weight, W, b):
    hidden = _matmul(x, W, b)
    pad = EP - E
    spread = jnp.arange(pad, dtype=jnp.int32) % N
    src = jnp.concatenate([edge_index[0], spread])
    dst = jnp.concatenate([edge_index[1], spread])
    w = jnp.concatenate([edge_weight, jnp.zeros((pad,), jnp.float32)])
    partial = _edge_pass(hidden, src, dst, w)
    return _epilogue(partial, x)


# speedup vs baseline: 3.6229x; 1.0225x over previous
import functools

import jax
import jax.numpy as jnp
from jax import lax
from jax.experimental import pallas as pl
from jax.experimental.pallas import tpu as pltpu
from jax.experimental.pallas import tpu_sc as plsc

N = 10000
E = 320000
D = 128

CHUNK = 128
NC, NS = 2, 16
NW = NC * NS
NUM_CHUNKS = E // CHUNK
CPT = 81
ZCHUNK = 80
NZ = N // ZCHUNK
ZITERS = (NZ + NS - 1) // NS


def _mm_kernel(x_ref, w_ref, b_ref, o_ref):
    o_ref[...] = (
        jnp.dot(x_ref[...], w_ref[...], preferred_element_type=jnp.float32)
        + b_ref[...]
    )


def _matmul(x, W, b):
    BN = 2000
    return pl.pallas_call(
        _mm_kernel,
        grid=(N // BN,),
        in_specs=[
            pl.BlockSpec((BN, D), lambda i: (i, 0)),
            pl.BlockSpec((D, D), lambda i: (0, 0)),
            pl.BlockSpec((1, D), lambda i: (0, 0)),
        ],
        out_specs=pl.BlockSpec((BN, D), lambda i: (i, 0)),
        out_shape=jax.ShapeDtypeStruct((N, D), jnp.float32),
    )(x, W, b.reshape(1, D))


def _ep_kernel(p_ref, x_ref, o_ref):
    o_ref[...] = jnp.maximum(p_ref[0] + p_ref[1], 0.0) + x_ref[...]


def _epilogue(partial, x):
    BN = 2000
    return pl.pallas_call(
        _ep_kernel,
        grid=(N // BN,),
        in_specs=[
            pl.BlockSpec((2, BN, D), lambda i: (0, i, 0)),
            pl.BlockSpec((BN, D), lambda i: (i, 0)),
        ],
        out_specs=pl.BlockSpec((BN, D), lambda i: (i, 0)),
        out_shape=jax.ShapeDtypeStruct((N, D), jnp.float32),
    )(partial, x)


def _scale_rows(rows, wbuf):
    def scale(g, c2):
        w16 = wbuf[pl.ds(g * 16, 16)]
        for k in range(16):
            s = w16[k]
            r = g * 16 + k
            for j in range(D // 16):
                sl = pl.ds(16 * j, 16)
                rows[r, sl] = rows[r, sl] * s
        return c2

    lax.fori_loop(0, CHUNK // 16, scale, 0, unroll=2)


def _edge_body(hidden_hbm, src_hbm, dst_hbm, w_hbm, partial_hbm,
               srcA, srcB, srcC, dstA, dstB, dstC, wA, wB, wC,
               rowsA, rowsB, rowsC, acc,
               semA, semB, semC, semSA, semSB, semSC):
    cid = lax.axis_index("c")
    sid = lax.axis_index("s")
    wid = sid * NC + cid

    zero = jnp.zeros((16,), jnp.float32)

    def zrow(r, carry):
        for j in range(D // 16):
            rowsA[r, pl.ds(16 * j, 16)] = zero
        return carry

    lax.fori_loop(0, ZCHUNK, zrow, 0)
    for k in range(ZITERS):
        c = sid + NS * k

        @pl.when(c < NZ)
        def _():
            pltpu.sync_copy(
                rowsA.at[pl.ds(0, ZCHUNK)],
                acc.at[pl.ds(c * ZCHUNK, ZCHUNK)],
            )

    plsc.subcore_barrier()

    def load_idx(chunk, src_v, dst_v, w_v, sem):
        base = chunk * CHUNK
        c1 = pltpu.async_copy(src_hbm.at[pl.ds(base, CHUNK)], src_v, sem)
        c2 = pltpu.async_copy(dst_hbm.at[pl.ds(base, CHUNK)], dst_v.at[0], sem)
        c3 = pltpu.async_copy(w_hbm.at[pl.ds(base, CHUNK)], w_v, sem)
        c1.wait()
        c2.wait()
        c3.wait()

    bufs = ((srcA, dstA, wA, rowsA, semA, semSA),
            (srcB, dstB, wB, rowsB, semB, semSB),
            (srcC, dstC, wC, rowsC, semC, semSC))

    load_idx(wid, srcA, dstA, wA, semA)
    pltpu.make_async_copy(hidden_hbm.at[srcA], rowsA, semA).start()
    load_idx(wid + NW, srcB, dstB, wB, semB)
    pltpu.make_async_copy(hidden_hbm.at[srcB], rowsB, semB).start()

    def body(g, carry):
        for o in range(3):
            j = 3 * g + o
            cj = wid + NW * j
            src_v, dst_v, w_v, rows, sem, semS = bufs[o]
            psrc, pdst, pw, prows, psem, psemS = bufs[(o + 2) % 3]

            @pl.when(cj < NUM_CHUNKS)
            def _():
                pltpu.make_async_copy(hidden_hbm.at[src_v], rows, sem).wait()
                _scale_rows(rows, w_v)
                pltpu.async_copy(rows, acc.at[dst_v.at[0]], semS, add=True)

            @pl.when(wid + NW * (j + 2) < NUM_CHUNKS)
            def _():
                def refill():
                    pltpu.make_async_copy(
                        prows, acc.at[pdst.at[0]], psemS).wait()
                    load_idx(wid + NW * (j + 2), psrc, pdst, pw, psem)
                    pltpu.make_async_copy(
                        hidden_hbm.at[psrc], prows, psem).start()

                if o == 0:
                    @pl.when(g > 0)
                    def _():
                        refill()

                    @pl.when(g == 0)
                    def _():
                        load_idx(wid + NW * (j + 2), psrc, pdst, pw, psem)
                        pltpu.make_async_copy(
                            hidden_hbm.at[psrc], prows, psem).start()
                else:
                    refill()

        return carry

    lax.fori_loop(0, CPT // 3, body, 0)
    for o in ((CPT - 3) % 3, (CPT - 2) % 3, (CPT - 1) % 3):
        _, dst_v, _, rows, _, semS = bufs[o]
        pltpu.make_async_copy(rows, acc.at[dst_v.at[0]], semS).wait()
    plsc.subcore_barrier()

    for k in range(ZITERS):
        c = sid + NS * k

        @pl.when(c < NZ)
        def _():
            pltpu.sync_copy(
                acc.at[pl.ds(c * ZCHUNK, ZCHUNK)],
                partial_hbm.at[cid, pl.ds(c * ZCHUNK, ZCHUNK)],
            )


def _edge_pass(hidden, src, dst, w):
    mesh = plsc.VectorSubcoreMesh(core_axis_name="c", subcore_axis_name="s")
    f = functools.partial(
        pl.kernel,
        mesh=mesh,
        out_type=jax.ShapeDtypeStruct((NC, N, D), jnp.float32),
        scratch_types=[
            pltpu.VMEM((CHUNK,), jnp.int32),
            pltpu.VMEM((CHUNK,), jnp.int32),
            pltpu.VMEM((CHUNK,), jnp.int32),
            pltpu.VMEM((1, CHUNK), jnp.int32),
            pltpu.VMEM((1, CHUNK), jnp.int32),
            pltpu.VMEM((1, CHUNK), jnp.int32),
            pltpu.VMEM((CHUNK,), jnp.float32),
            pltpu.VMEM((CHUNK,), jnp.float32),
            pltpu.VMEM((CHUNK,), jnp.float32),
            pltpu.VMEM((CHUNK, D), jnp.float32),
            pltpu.VMEM((CHUNK, D), jnp.float32),
            pltpu.VMEM((CHUNK, D), jnp.float32),
            pltpu.VMEM_SHARED((N, D), jnp.float32),
            pltpu.SemaphoreType.DMA,
            pltpu.SemaphoreType.DMA,
            pltpu.SemaphoreType.DMA,
            pltpu.SemaphoreType.DMA,
            pltpu.SemaphoreType.DMA,
            pltpu.SemaphoreType.DMA,
        ],
    )(_edge_body)
    return f(hidden, src, dst, w)


def kernel(x, edge_index, edge_weight, W, b):
    hidden = _matmul(x, W, b)
    partial = _edge_pass(hidden, edge_index[0], edge_index[1], edge_weight)
    return _epilogue(partial, x)


# speedup vs baseline: 3.9651x; 1.0944x over previous
import functools

import jax
import jax.numpy as jnp
from jax import lax
from jax.experimental import pallas as pl
from jax.experimental.pallas import tpu as pltpu
from jax.experimental.pallas import tpu_sc as plsc

N = 10000
E = 320000
D = 128

CHUNK = 128
NC, NS = 2, 16
NW = NC * NS
NUM_CHUNKS = E // CHUNK
CPT = 81
ZCHUNK = 80
NZ = N // ZCHUNK
ZITERS = (NZ + NS - 1) // NS


def _mm_kernel(x_ref, w_ref, b_ref, o_ref):
    o_ref[...] = (
        jnp.dot(x_ref[...], w_ref[...], preferred_element_type=jnp.float32)
        + b_ref[...]
    )


def _matmul(x, W, b):
    BN = 2000
    return pl.pallas_call(
        _mm_kernel,
        grid=(N // BN,),
        in_specs=[
            pl.BlockSpec((BN, D), lambda i: (i, 0)),
            pl.BlockSpec((D, D), lambda i: (0, 0)),
            pl.BlockSpec((1, D), lambda i: (0, 0)),
        ],
        out_specs=pl.BlockSpec((BN, D), lambda i: (i, 0)),
        out_shape=jax.ShapeDtypeStruct((N, D), jnp.float32),
    )(x, W, b.reshape(1, D))


def _ep_kernel(p_ref, x_ref, o_ref):
    o_ref[...] = jnp.maximum(p_ref[0] + p_ref[1], 0.0) + x_ref[...]


def _epilogue(partial, x):
    BN = 2000
    return pl.pallas_call(
        _ep_kernel,
        grid=(N // BN,),
        in_specs=[
            pl.BlockSpec((2, BN, D), lambda i: (0, i, 0)),
            pl.BlockSpec((BN, D), lambda i: (i, 0)),
        ],
        out_specs=pl.BlockSpec((BN, D), lambda i: (i, 0)),
        out_shape=jax.ShapeDtypeStruct((N, D), jnp.float32),
    )(partial, x)


def _scale_rows(rows, wbuf):
    def scale(g, c2):
        w16 = wbuf[pl.ds(g * 16, 16)]
        for k in range(16):
            s = w16[k]
            r = g * 16 + k
            for j in range(D // 16):
                sl = pl.ds(16 * j, 16)
                rows[r, sl] = rows[r, sl] * s
        return c2

    lax.fori_loop(0, CHUNK // 16, scale, 0, unroll=2)


def _edge_body(hidden_hbm, ei_hbm, w_hbm, partial_hbm,
               srcA, srcB, srcC, dstA, dstB, dstC, wA, wB, wC,
               rowsA, rowsB, rowsC, acc,
               semA, semB, semC, semSA, semSB, semSC):
    cid = lax.axis_index("c")
    sid = lax.axis_index("s")
    wid = sid * NC + cid

    zero = jnp.zeros((16,), jnp.float32)

    def zrow(r, carry):
        for j in range(D // 16):
            rowsA[r, pl.ds(16 * j, 16)] = zero
        return carry

    lax.fori_loop(0, ZCHUNK, zrow, 0)
    for k in range(ZITERS):
        c = sid + NS * k

        @pl.when(c < NZ)
        def _():
            pltpu.sync_copy(
                rowsA.at[pl.ds(0, ZCHUNK)],
                acc.at[pl.ds(c * ZCHUNK, ZCHUNK)],
            )

    plsc.subcore_barrier()

    def load_idx(chunk, src_v, dst_v, w_v, sem):
        base = chunk * CHUNK
        c1 = pltpu.async_copy(ei_hbm.at[0, pl.ds(base, CHUNK)], src_v, sem)
        c2 = pltpu.async_copy(ei_hbm.at[1, pl.ds(base, CHUNK)], dst_v.at[0], sem)
        c3 = pltpu.async_copy(w_hbm.at[pl.ds(base, CHUNK)], w_v, sem)
        c1.wait()
        c2.wait()
        c3.wait()

    bufs = ((srcA, dstA, wA, rowsA, semA, semSA),
            (srcB, dstB, wB, rowsB, semB, semSB),
            (srcC, dstC, wC, rowsC, semC, semSC))

    load_idx(wid, srcA, dstA, wA, semA)
    pltpu.make_async_copy(hidden_hbm.at[srcA], rowsA, semA).start()
    load_idx(wid + NW, srcB, dstB, wB, semB)
    pltpu.make_async_copy(hidden_hbm.at[srcB], rowsB, semB).start()

    def body(g, carry):
        for o in range(3):
            j = 3 * g + o
            cj = wid + NW * j
            src_v, dst_v, w_v, rows, sem, semS = bufs[o]
            psrc, pdst, pw, prows, psem, psemS = bufs[(o + 2) % 3]

            @pl.when(cj < NUM_CHUNKS)
            def _():
                pltpu.make_async_copy(hidden_hbm.at[src_v], rows, sem).wait()
                _scale_rows(rows, w_v)
                pltpu.async_copy(rows, acc.at[dst_v.at[0]], semS, add=True)

            @pl.when(wid + NW * (j + 2) < NUM_CHUNKS)
            def _():
                def refill():
                    pltpu.make_async_copy(
                        prows, acc.at[pdst.at[0]], psemS).wait()
                    load_idx(wid + NW * (j + 2), psrc, pdst, pw, psem)
                    pltpu.make_async_copy(
                        hidden_hbm.at[psrc], prows, psem).start()

                if o == 0:
                    @pl.when(g > 0)
                    def _():
                        refill()

                    @pl.when(g == 0)
                    def _():
                        load_idx(wid + NW * (j + 2), psrc, pdst, pw, psem)
                        pltpu.make_async_copy(
                            hidden_hbm.at[psrc], prows, psem).start()
                else:
                    refill()

        return carry

    lax.fori_loop(0, CPT // 3, body, 0)
    for o in ((CPT - 3) % 3, (CPT - 2) % 3, (CPT - 1) % 3):
        _, dst_v, _, rows, _, semS = bufs[o]
        pltpu.make_async_copy(rows, acc.at[dst_v.at[0]], semS).wait()
    plsc.subcore_barrier()

    for k in range(ZITERS):
        c = sid + NS * k

        @pl.when(c < NZ)
        def _():
            pltpu.sync_copy(
                acc.at[pl.ds(c * ZCHUNK, ZCHUNK)],
                partial_hbm.at[cid, pl.ds(c * ZCHUNK, ZCHUNK)],
            )


def _edge_pass(hidden, ei, w):
    mesh = plsc.VectorSubcoreMesh(core_axis_name="c", subcore_axis_name="s")
    f = functools.partial(
        pl.kernel,
        mesh=mesh,
        out_type=jax.ShapeDtypeStruct((NC, N, D), jnp.float32),
        scratch_types=[
            pltpu.VMEM((CHUNK,), jnp.int32),
            pltpu.VMEM((CHUNK,), jnp.int32),
            pltpu.VMEM((CHUNK,), jnp.int32),
            pltpu.VMEM((1, CHUNK), jnp.int32),
            pltpu.VMEM((1, CHUNK), jnp.int32),
            pltpu.VMEM((1, CHUNK), jnp.int32),
            pltpu.VMEM((CHUNK,), jnp.float32),
            pltpu.VMEM((CHUNK,), jnp.float32),
            pltpu.VMEM((CHUNK,), jnp.float32),
            pltpu.VMEM((CHUNK, D), jnp.float32),
            pltpu.VMEM((CHUNK, D), jnp.float32),
            pltpu.VMEM((CHUNK, D), jnp.float32),
            pltpu.VMEM_SHARED((N, D), jnp.float32),
            pltpu.SemaphoreType.DMA,
            pltpu.SemaphoreType.DMA,
            pltpu.SemaphoreType.DMA,
            pltpu.SemaphoreType.DMA,
            pltpu.SemaphoreType.DMA,
            pltpu.SemaphoreType.DMA,
        ],
    )(_edge_body)
    return f(hidden, ei, w)


def kernel(x, edge_index, edge_weight, W, b):
    hidden = _matmul(x, W, b)
    partial = _edge_pass(hidden, edge_index, edge_weight)
    return _epilogue(partial, x)


# speedup vs baseline: 4.0124x; 1.0119x over previous
import functools

import jax
import jax.numpy as jnp
from jax import lax
from jax.experimental import pallas as pl
from jax.experimental.pallas import tpu as pltpu
from jax.experimental.pallas import tpu_sc as plsc

N = 10000
E = 320000
D = 128

CHUNK = 128
NC, NS = 2, 16
NW = NC * NS
NUM_CHUNKS = E // CHUNK
CPT = 81
ZCHUNK = 80
NZ = N // ZCHUNK
ZITERS = (NZ + NS - 1) // NS


def _mm_kernel(x_ref, w_ref, b_ref, o_ref):
    o_ref[...] = (
        jnp.dot(x_ref[...], w_ref[...], preferred_element_type=jnp.float32)
        + b_ref[...]
    )


def _matmul(x, W, b):
    BN = 2000
    return pl.pallas_call(
        _mm_kernel,
        grid=(N // BN,),
        in_specs=[
            pl.BlockSpec((BN, D), lambda i: (i, 0)),
            pl.BlockSpec((D, D), lambda i: (0, 0)),
            pl.BlockSpec((1, D), lambda i: (0, 0)),
        ],
        out_specs=pl.BlockSpec((BN, D), lambda i: (i, 0)),
        out_shape=jax.ShapeDtypeStruct((N, D), jnp.float32),
    )(x, W, b.reshape(1, D))


def _ep_kernel(p_ref, x_ref, o_ref):
    o_ref[...] = jnp.maximum(p_ref[0] + p_ref[1], 0.0) + x_ref[...]


def _epilogue(partial, x):
    BN = 2000
    return pl.pallas_call(
        _ep_kernel,
        grid=(N // BN,),
        in_specs=[
            pl.BlockSpec((2, BN, D), lambda i: (0, i, 0)),
            pl.BlockSpec((BN, D), lambda i: (i, 0)),
        ],
        out_specs=pl.BlockSpec((BN, D), lambda i: (i, 0)),
        out_shape=jax.ShapeDtypeStruct((N, D), jnp.float32),
    )(partial, x)


def _scale_rows(rows, wbuf):
    def scale(g, c2):
        w16 = wbuf[pl.ds(g * 16, 16)]
        for k in range(16):
            s = w16[k]
            r = g * 16 + k
            for j in range(D // 16):
                sl = pl.ds(16 * j, 16)
                rows[r, sl] = rows[r, sl] * s
        return c2

    lax.fori_loop(0, CHUNK // 16, scale, 0, unroll=2)


def _edge_body(hidden_hbm, ei_hbm, w_hbm, partial_hbm,
               srcA, srcB, srcC, dstA, dstB, dstC, wA, wB, wC,
               rowsA, rowsB, rowsC, acc,
               semA, semB, semC, semSA, semSB, semSC):
    cid = lax.axis_index("c")
    sid = lax.axis_index("s")
    wid = sid * NC + cid

    def load_idx(chunk, src_v, dst_v, w_v, sem):
        base = chunk * CHUNK
        c1 = pltpu.async_copy(ei_hbm.at[0, pl.ds(base, CHUNK)], src_v, sem)
        c2 = pltpu.async_copy(ei_hbm.at[1, pl.ds(base, CHUNK)], dst_v.at[0], sem)
        c3 = pltpu.async_copy(w_hbm.at[pl.ds(base, CHUNK)], w_v, sem)
        c1.wait()
        c2.wait()
        c3.wait()

    load_idx(wid, srcA, dstA, wA, semA)
    pltpu.make_async_copy(hidden_hbm.at[srcA], rowsA, semA).start()
    load_idx(wid + NW, srcB, dstB, wB, semB)
    pltpu.make_async_copy(hidden_hbm.at[srcB], rowsB, semB).start()

    zero = jnp.zeros((16,), jnp.float32)

    def zrow(r, carry):
        for j in range(D // 16):
            rowsC[r, pl.ds(16 * j, 16)] = zero
        return carry

    lax.fori_loop(0, ZCHUNK, zrow, 0)
    for k in range(ZITERS):
        c = sid + NS * k

        @pl.when(c < NZ)
        def _():
            pltpu.sync_copy(
                rowsC.at[pl.ds(0, ZCHUNK)],
                acc.at[pl.ds(c * ZCHUNK, ZCHUNK)],
            )

    plsc.subcore_barrier()

    bufs = ((srcA, dstA, wA, rowsA, semA, semSA),
            (srcB, dstB, wB, rowsB, semB, semSB),
            (srcC, dstC, wC, rowsC, semC, semSC))


    def body(g, carry):
        for o in range(3):
            j = 3 * g + o
            cj = wid + NW * j
            src_v, dst_v, w_v, rows, sem, semS = bufs[o]
            psrc, pdst, pw, prows, psem, psemS = bufs[(o + 2) % 3]

            @pl.when(cj < NUM_CHUNKS)
            def _():
                pltpu.make_async_copy(hidden_hbm.at[src_v], rows, sem).wait()
                _scale_rows(rows, w_v)
                pltpu.async_copy(rows, acc.at[dst_v.at[0]], semS, add=True)

            @pl.when(wid + NW * (j + 2) < NUM_CHUNKS)
            def _():
                def refill():
                    pltpu.make_async_copy(
                        prows, acc.at[pdst.at[0]], psemS).wait()
                    load_idx(wid + NW * (j + 2), psrc, pdst, pw, psem)
                    pltpu.make_async_copy(
                        hidden_hbm.at[psrc], prows, psem).start()

                if o == 0:
                    @pl.when(g > 0)
                    def _():
                        refill()

                    @pl.when(g == 0)
                    def _():
                        load_idx(wid + NW * (j + 2), psrc, pdst, pw, psem)
                        pltpu.make_async_copy(
                            hidden_hbm.at[psrc], prows, psem).start()
                else:
                    refill()

        return carry

    lax.fori_loop(0, CPT // 3, body, 0)
    for o in ((CPT - 3) % 3, (CPT - 2) % 3, (CPT - 1) % 3):
        _, dst_v, _, rows, _, semS = bufs[o]
        pltpu.make_async_copy(rows, acc.at[dst_v.at[0]], semS).wait()
    plsc.subcore_barrier()

    for k in range(ZITERS):
        c = sid + NS * k

        @pl.when(c < NZ)
        def _():
            pltpu.sync_copy(
                acc.at[pl.ds(c * ZCHUNK, ZCHUNK)],
                partial_hbm.at[cid, pl.ds(c * ZCHUNK, ZCHUNK)],
            )


def _edge_pass(hidden, ei, w):
    mesh = plsc.VectorSubcoreMesh(core_axis_name="c", subcore_axis_name="s")
    f = functools.partial(
        pl.kernel,
        mesh=mesh,
        out_type=jax.ShapeDtypeStruct((NC, N, D), jnp.float32),
        scratch_types=[
            pltpu.VMEM((CHUNK,), jnp.int32),
            pltpu.VMEM((CHUNK,), jnp.int32),
            pltpu.VMEM((CHUNK,), jnp.int32),
            pltpu.VMEM((1, CHUNK), jnp.int32),
            pltpu.VMEM((1, CHUNK), jnp.int32),
            pltpu.VMEM((1, CHUNK), jnp.int32),
            pltpu.VMEM((CHUNK,), jnp.float32),
            pltpu.VMEM((CHUNK,), jnp.float32),
            pltpu.VMEM((CHUNK,), jnp.float32),
            pltpu.VMEM((CHUNK, D), jnp.float32),
            pltpu.VMEM((CHUNK, D), jnp.float32),
            pltpu.VMEM((CHUNK, D), jnp.float32),
            pltpu.VMEM_SHARED((N, D), jnp.float32),
            pltpu.SemaphoreType.DMA,
            pltpu.SemaphoreType.DMA,
            pltpu.SemaphoreType.DMA,
            pltpu.SemaphoreType.DMA,
            pltpu.SemaphoreType.DMA,
            pltpu.SemaphoreType.DMA,
        ],
    )(_edge_body)
    return f(hidden, ei, w)


def kernel(x, edge_index, edge_weight, W, b):
    hidden = _matmul(x, W, b)
    partial = _edge_pass(hidden, edge_index, edge_weight)
    return _epilogue(partial, x)
